# baseline (speedup 1.0000x reference)
import functools

import jax
import jax.numpy as jnp
from jax import lax
from jax.experimental import pallas as pl
from jax.experimental.pallas import tpu as pltpu
from jax.experimental.pallas import tpu_sc as plsc

N = 10000
E = 320000
D = 128
L = 3

NC, NS, LANES = 2, 16, 16
NW = NC * NS
N_PAD = 10240
EC = 64
CH = 160
EPT = CH * EC
E_PAD = NW * EPT
ROWS_PT = N_PAD // NS
TOK_CH = 4
RB = 1024

_sc_mesh = plsc.VectorSubcoreMesh(core_axis_name="c", subcore_axis_name="s")


@functools.partial(
    pl.kernel,
    out_type=jax.ShapeDtypeStruct((N_PAD, D), jnp.float32),
    mesh=_sc_mesh,
    scratch_types=[
        pltpu.VMEM((TOK_CH, 80), jnp.int32),
        pltpu.VMEM((80, D), jnp.float32),
        pltpu.SemaphoreType.DMA,
    ],
)
def _emb_gather(emb_hbm, tok_hbm, out_hbm, idx_v, rows_v, sem):
    cid = lax.axis_index("c")
    sid = lax.axis_index("s")
    wid = sid * NC + cid
    pltpu.sync_copy(tok_hbm.at[wid], idx_v)
    base = wid * (TOK_CH * 80)

    def chunk(ch, _):
        pltpu.async_copy(emb_hbm.at[idx_v.at[ch]], rows_v, sem).wait()
        pltpu.sync_copy(rows_v, out_hbm.at[pl.ds(base + ch * 80, 80)])
        return 0

    lax.fori_loop(0, TOK_CH, chunk, 0)


@functools.partial(
    pl.kernel,
    out_type=jax.ShapeDtypeStruct((NC, N_PAD), jnp.float32),
    mesh=_sc_mesh,
    scratch_types=[
        pltpu.VMEM((CH, EC), jnp.int32),
        pltpu.VMEM((N_PAD,), jnp.float32),
        pltpu.VMEM((ROWS_PT,), jnp.float32),
        pltpu.VMEM((ROWS_PT,), jnp.float32),
        pltpu.VMEM_SHARED((NS, N_PAD), jnp.float32),
    ],
    compiler_params=pltpu.CompilerParams(needs_layout_passes=False),
)
def _deg_pass(dst_hbm, out_hbm, dst_v, degloc, acc, tmp, sdeg):
    cid = lax.axis_index("c")
    sid = lax.axis_index("s")
    pltpu.sync_copy(dst_hbm.at[cid, sid], dst_v)

    zvec = jnp.zeros((LANES,), jnp.float32)
    ones = jnp.ones((LANES,), jnp.float32)

    def z(i, _):
        degloc[pl.ds(i * LANES, LANES)] = zvec
        return 0

    lax.fori_loop(0, N_PAD // LANES, z, 0)

    def acc_chunk(ch, _):
        for k in range(EC // LANES):
            idx = dst_v[ch, pl.ds(k * LANES, LANES)]
            plsc.addupdate_scatter(degloc, [idx], ones)
        return 0

    lax.fori_loop(0, CH, acc_chunk, 0)

    pltpu.sync_copy(degloc, sdeg.at[sid])
    plsc.subcore_barrier()

    base = sid * ROWS_PT

    def z2(i, _):
        acc[pl.ds(i * LANES, LANES)] = zvec
        return 0

    lax.fori_loop(0, ROWS_PT // LANES, z2, 0)
    for p in range(NS):
        pltpu.sync_copy(sdeg.at[p, pl.ds(base, ROWS_PT)], tmp)

        def addp(j, _):
            sl = pl.ds(j * LANES, LANES)
            acc[sl] = acc[sl] + tmp[sl]
            return 0

        lax.fori_loop(0, ROWS_PT // LANES, addp, 0)
    pltpu.sync_copy(acc, out_hbm.at[cid, pl.ds(base, ROWS_PT)])


@functools.partial(
    pl.kernel,
    out_type=jax.ShapeDtypeStruct((NC, N_PAD, D), jnp.float32),
    mesh=_sc_mesh,
    scratch_types=[
        pltpu.VMEM((CH // 2 + 1, 128), jnp.int32),
        pltpu.VMEM((CH, EC), jnp.int32),
        pltpu.VMEM((EC, D), jnp.float32),
        pltpu.VMEM((EC, D), jnp.float32),
        pltpu.VMEM_SHARED((N_PAD, D), jnp.float32),
        pltpu.SemaphoreType.DMA,
        pltpu.SemaphoreType.DMA,
        pltpu.SemaphoreType.DMA,
        pltpu.SemaphoreType.DMA,
    ],
)
def _edge_pass(t_hbm, src_hbm, dst_hbm, out_hbm,
               src_v, dst_v, buf_a, buf_b, agg_sh, semg_a, semg_b,
               sems_a, sems_b):
    cid = lax.axis_index("c")
    sid = lax.axis_index("s")
    pltpu.sync_copy(src_hbm.at[cid, sid], src_v.at[pl.ds(0, CH // 2)])
    pltpu.sync_copy(dst_hbm.at[cid, sid], dst_v)

    zvec = jnp.zeros((LANES,), jnp.float32)
    izvec = jnp.zeros((LANES,), jnp.int32)
    for k in range(128 // LANES):
        src_v[CH // 2, pl.ds(k * LANES, LANES)] = izvec

    def zr(i, _):
        for k in range(D // LANES):
            buf_a[i, pl.ds(k * LANES, LANES)] = zvec
        return 0

    lax.fori_loop(0, EC, zr, 0)
    for k in range(ROWS_PT // EC):
        pltpu.sync_copy(buf_a, agg_sh.at[pl.ds(sid * ROWS_PT + k * EC, EC)])
    plsc.subcore_barrier()

    pltpu.async_copy(t_hbm.at[src_v.at[0, pl.ds(0, EC)]], buf_a, semg_a)
    pltpu.async_copy(t_hbm.at[src_v.at[0, pl.ds(EC, EC)]], buf_b, semg_b)

    def body(i, _):
        ch = i * 2
        pltpu.make_async_copy(
            t_hbm.at[src_v.at[0, pl.ds(0, EC)]], buf_a, semg_a).wait()
        pltpu.async_copy(buf_a, agg_sh.at[dst_v.at[ch]], sems_a, add=True)
        pltpu.make_async_copy(
            t_hbm.at[src_v.at[0, pl.ds(EC, EC)]], buf_b, semg_b).wait()
        pltpu.async_copy(buf_b, agg_sh.at[dst_v.at[ch + 1]], sems_b, add=True)
        pltpu.make_async_copy(
            buf_a, agg_sh.at[dst_v.at[0]], sems_a).wait()
        pltpu.async_copy(
            t_hbm.at[src_v.at[i + 1, pl.ds(0, EC)]], buf_a, semg_a)
        pltpu.make_async_copy(
            buf_b, agg_sh.at[dst_v.at[0]], sems_b).wait()
        pltpu.async_copy(
            t_hbm.at[src_v.at[i + 1, pl.ds(EC, EC)]], buf_b, semg_b)
        return 0

    lax.fori_loop(0, CH // 2, body, 0)
    pltpu.make_async_copy(
        t_hbm.at[src_v.at[0, pl.ds(0, EC)]], buf_a, semg_a).wait()
    pltpu.make_async_copy(
        t_hbm.at[src_v.at[0, pl.ds(EC, EC)]], buf_b, semg_b).wait()
    plsc.subcore_barrier()
    pltpu.sync_copy(agg_sh.at[pl.ds(sid * ROWS_PT, ROWS_PT)],
                    out_hbm.at[cid, pl.ds(sid * ROWS_PT, ROWS_PT)])


def _mm0_body(x_ref, wc_ref, bc_ref, ws_ref, h0_ref, t_ref):
    h0 = jnp.dot(x_ref[...], wc_ref[...],
                 preferred_element_type=jnp.float32) + bc_ref[...]
    h0_ref[...] = h0
    t_ref[...] = jnp.dot(h0, ws_ref[...], preferred_element_type=jnp.float32)


_mm0 = pl.pallas_call(
    _mm0_body,
    grid=(N_PAD // RB,),
    in_specs=[
        pl.BlockSpec((RB, D), lambda i: (i, 0)),
        pl.BlockSpec((D, D), lambda i: (0, 0)),
        pl.BlockSpec((1, D), lambda i: (0, 0)),
        pl.BlockSpec((D, D), lambda i: (0, 0)),
    ],
    out_specs=[
        pl.BlockSpec((RB, D), lambda i: (i, 0)),
        pl.BlockSpec((RB, D), lambda i: (i, 0)),
    ],
    out_shape=[
        jax.ShapeDtypeStruct((N_PAD, D), jnp.float32),
        jax.ShapeDtypeStruct((N_PAD, D), jnp.float32),
    ],
)


def _combine_h(h0, p, deg, bs):
    agg = p[0] + p[1]
    w = 1.0 / jnp.maximum(deg, 1.0)
    gate = jnp.minimum(deg, 1.0)
    return h0 + agg * w + gate * bs


def _combine_body(h0_ref, p_ref, deg_ref, bs_ref, ws_ref, r_ref, t_ref):
    h = _combine_h(h0_ref[...], p_ref[...], deg_ref[...], bs_ref[...])
    r_ref[...] = jnp.maximum(h, 0.0)
    t_ref[...] = jnp.dot(h, ws_ref[...], preferred_element_type=jnp.float32)


_combine = pl.pallas_call(
    _combine_body,
    grid=(N_PAD // RB,),
    in_specs=[
        pl.BlockSpec((RB, D), lambda i: (i, 0)),
        pl.BlockSpec((NC, RB, D), lambda i: (0, i, 0)),
        pl.BlockSpec((RB, 1), lambda i: (i, 0)),
        pl.BlockSpec((1, D), lambda i: (0, 0)),
        pl.BlockSpec((D, D), lambda i: (0, 0)),
    ],
    out_specs=[
        pl.BlockSpec((RB, D), lambda i: (i, 0)),
        pl.BlockSpec((RB, D), lambda i: (i, 0)),
    ],
    out_shape=[
        jax.ShapeDtypeStruct((N_PAD, D), jnp.float32),
        jax.ShapeDtypeStruct((N_PAD, D), jnp.float32),
    ],
)


def _combine3_body(h0_ref, p_ref, deg_ref, bs_ref, r1_ref, r2_ref,
                   r3_ref, m_ref):
    h = _combine_h(h0_ref[...], p_ref[...], deg_ref[...], bs_ref[...])
    r3 = jnp.maximum(h, 0.0)
    r3_ref[...] = r3
    m_ref[...] = jnp.maximum(jnp.maximum(r1_ref[...], r2_ref[...]), r3)


_combine3 = pl.pallas_call(
    _combine3_body,
    grid=(N_PAD // RB,),
    in_specs=[
        pl.BlockSpec((RB, D), lambda i: (i, 0)),
        pl.BlockSpec((NC, RB, D), lambda i: (0, i, 0)),
        pl.BlockSpec((RB, 1), lambda i: (i, 0)),
        pl.BlockSpec((1, D), lambda i: (0, 0)),
        pl.BlockSpec((RB, D), lambda i: (i, 0)),
        pl.BlockSpec((RB, D), lambda i: (i, 0)),
    ],
    out_specs=[
        pl.BlockSpec((RB, D), lambda i: (i, 0)),
        pl.BlockSpec((RB, D), lambda i: (i, 0)),
    ],
    out_shape=[
        jax.ShapeDtypeStruct((N_PAD, D), jnp.float32),
        jax.ShapeDtypeStruct((N_PAD, D), jnp.float32),
    ],
)


def kernel(node_tokens, edge_index, emb, W_c, b_c, W_sum, b_sum):
    tok = jnp.pad(node_tokens.astype(jnp.int32), (0, N_PAD - N))
    tok = tok.reshape(NW, TOK_CH, 80)
    pad_e = E_PAD - E
    src = jnp.concatenate(
        [edge_index[0].astype(jnp.int32), jnp.zeros((pad_e,), jnp.int32)])
    dst = jnp.concatenate(
        [edge_index[1].astype(jnp.int32), jnp.full((pad_e,), N, jnp.int32)])
    src_r = src.reshape(NC, NS, CH // 2, 128)
    dst_r = dst.reshape(NC, NS, CH, EC)
    bc2 = b_c.reshape(1, D)
    bs2 = b_sum.reshape(1, D)

    x = _emb_gather(emb, tok)
    degp = _deg_pass(dst_r)
    deg2 = (degp[0] + degp[1]).reshape(N_PAD, 1)
    h0, t = _mm0(x, W_c, bc2, W_sum)
    p = _edge_pass(t, src_r, dst_r)
    r1, t = _combine(h0, p, deg2, bs2, W_sum)
    p = _edge_pass(t, src_r, dst_r)
    r2, t = _combine(h0, p, deg2, bs2, W_sum)
    p = _edge_pass(t, src_r, dst_r)
    r3, m = _combine3(h0, p, deg2, bs2, r1, r2)

    nl = jnp.stack([r1[:N], r2[:N], r3[:N]])
    return nl, m[:N]


# speedup vs baseline: 1.3871x; 1.3871x over previous
import functools

import jax
import jax.numpy as jnp
from jax import lax
from jax.experimental import pallas as pl
from jax.experimental.pallas import tpu as pltpu
from jax.experimental.pallas import tpu_sc as plsc

N = 10000
E = 320000
D = 128
L = 3

NC, NS, LANES = 2, 16, 16
NW = NC * NS
N_PAD = 10240
EC = 128
CH = 80
EPT = CH * EC
E_PAD = NW * EPT
ROWS_PT = N_PAD // NS
TOK_CH = 4
RB = 1024

_sc_mesh = plsc.VectorSubcoreMesh(core_axis_name="c", subcore_axis_name="s")


@functools.partial(
    pl.kernel,
    out_type=jax.ShapeDtypeStruct((N_PAD, D), jnp.float32),
    mesh=_sc_mesh,
    scratch_types=[
        pltpu.VMEM((TOK_CH, 80), jnp.int32),
        pltpu.VMEM((80, D), jnp.float32),
        pltpu.SemaphoreType.DMA,
    ],
)
def _emb_gather(emb_hbm, tok_hbm, out_hbm, idx_v, rows_v, sem):
    cid = lax.axis_index("c")
    sid = lax.axis_index("s")
    wid = sid * NC + cid
    pltpu.sync_copy(tok_hbm.at[wid], idx_v)
    base = wid * (TOK_CH * 80)

    def chunk(ch, _):
        pltpu.async_copy(emb_hbm.at[idx_v.at[ch]], rows_v, sem).wait()
        pltpu.sync_copy(rows_v, out_hbm.at[pl.ds(base + ch * 80, 80)])
        return 0

    lax.fori_loop(0, TOK_CH, chunk, 0)


@functools.partial(
    pl.kernel,
    out_type=jax.ShapeDtypeStruct((NC, N_PAD), jnp.float32),
    mesh=_sc_mesh,
    scratch_types=[
        pltpu.VMEM((CH, EC), jnp.int32),
        pltpu.VMEM((N_PAD,), jnp.float32),
        pltpu.VMEM((ROWS_PT,), jnp.float32),
        pltpu.VMEM((ROWS_PT,), jnp.float32),
        pltpu.VMEM_SHARED((NS, N_PAD), jnp.float32),
    ],
    compiler_params=pltpu.CompilerParams(needs_layout_passes=False),
)
def _deg_pass(dst_hbm, out_hbm, dst_v, degloc, acc, tmp, sdeg):
    cid = lax.axis_index("c")
    sid = lax.axis_index("s")
    pltpu.sync_copy(dst_hbm.at[cid, sid], dst_v)

    zvec = jnp.zeros((LANES,), jnp.float32)
    ones = jnp.ones((LANES,), jnp.float32)

    def z(i, _):
        degloc[pl.ds(i * LANES, LANES)] = zvec
        return 0

    lax.fori_loop(0, N_PAD // LANES, z, 0)

    def acc_chunk(ch, _):
        for k in range(EC // LANES):
            idx = dst_v[ch, pl.ds(k * LANES, LANES)]
            plsc.addupdate_scatter(degloc, [idx], ones)
        return 0

    lax.fori_loop(0, CH, acc_chunk, 0)

    pltpu.sync_copy(degloc, sdeg.at[sid])
    plsc.subcore_barrier()

    base = sid * ROWS_PT

    def z2(i, _):
        acc[pl.ds(i * LANES, LANES)] = zvec
        return 0

    lax.fori_loop(0, ROWS_PT // LANES, z2, 0)
    for p in range(NS):
        pltpu.sync_copy(sdeg.at[p, pl.ds(base, ROWS_PT)], tmp)

        def addp(j, _):
            sl = pl.ds(j * LANES, LANES)
            acc[sl] = acc[sl] + tmp[sl]
            return 0

        lax.fori_loop(0, ROWS_PT // LANES, addp, 0)
    pltpu.sync_copy(acc, out_hbm.at[cid, pl.ds(base, ROWS_PT)])


@functools.partial(
    pl.kernel,
    out_type=jax.ShapeDtypeStruct((NC, N_PAD, D), jnp.float32),
    mesh=_sc_mesh,
    scratch_types=[
        pltpu.VMEM((CH, EC), jnp.int32),
        pltpu.VMEM((CH, EC), jnp.int32),
        pltpu.VMEM((EC, D), jnp.float32),
        pltpu.VMEM_SHARED((N_PAD, D), jnp.float32),
        pltpu.SemaphoreType.DMA,
        pltpu.SemaphoreType.DMA,
    ],
)
def _edge_pass(t_hbm, src_hbm, dst_hbm, out_hbm,
               src_v, dst_v, buf_a, agg_sh, semg_a, sems_a):
    cid = lax.axis_index("c")
    sid = lax.axis_index("s")
    pltpu.sync_copy(src_hbm.at[cid, sid], src_v)
    pltpu.sync_copy(dst_hbm.at[cid, sid], dst_v)

    zvec = jnp.zeros((LANES,), jnp.float32)

    def zr(i, _):
        for k in range(D // LANES):
            buf_a[i, pl.ds(k * LANES, LANES)] = zvec
        return 0

    lax.fori_loop(0, EC, zr, 0)
    for k in range(ROWS_PT // EC):
        pltpu.sync_copy(buf_a, agg_sh.at[pl.ds(sid * ROWS_PT + k * EC, EC)])
    plsc.subcore_barrier()

    def body(i, _):
        pltpu.async_copy(t_hbm.at[src_v.at[i]], buf_a, semg_a).wait()
        pltpu.async_copy(buf_a, agg_sh.at[dst_v.at[i]], sems_a,
                         add=True).wait()
        return 0

    lax.fori_loop(0, CH, body, 0)
    plsc.subcore_barrier()
    pltpu.sync_copy(agg_sh.at[pl.ds(sid * ROWS_PT, ROWS_PT)],
                    out_hbm.at[cid, pl.ds(sid * ROWS_PT, ROWS_PT)])


def _mm0_body(x_ref, wc_ref, bc_ref, ws_ref, h0_ref, t_ref):
    h0 = jnp.dot(x_ref[...], wc_ref[...],
                 preferred_element_type=jnp.float32) + bc_ref[...]
    h0_ref[...] = h0
    t_ref[...] = jnp.dot(h0, ws_ref[...], preferred_element_type=jnp.float32)


_mm0 = pl.pallas_call(
    _mm0_body,
    grid=(N_PAD // RB,),
    in_specs=[
        pl.BlockSpec((RB, D), lambda i: (i, 0)),
        pl.BlockSpec((D, D), lambda i: (0, 0)),
        pl.BlockSpec((1, D), lambda i: (0, 0)),
        pl.BlockSpec((D, D), lambda i: (0, 0)),
    ],
    out_specs=[
        pl.BlockSpec((RB, D), lambda i: (i, 0)),
        pl.BlockSpec((RB, D), lambda i: (i, 0)),
    ],
    out_shape=[
        jax.ShapeDtypeStruct((N_PAD, D), jnp.float32),
        jax.ShapeDtypeStruct((N_PAD, D), jnp.float32),
    ],
)


def _combine_h(h0, p, deg, bs):
    agg = p[0] + p[1]
    w = 1.0 / jnp.maximum(deg, 1.0)
    gate = jnp.minimum(deg, 1.0)
    return h0 + agg * w + gate * bs


def _combine_body(h0_ref, p_ref, deg_ref, bs_ref, ws_ref, r_ref, t_ref):
    h = _combine_h(h0_ref[...], p_ref[...], deg_ref[...], bs_ref[...])
    r_ref[...] = jnp.maximum(h, 0.0)
    t_ref[...] = jnp.dot(h, ws_ref[...], preferred_element_type=jnp.float32)


_combine = pl.pallas_call(
    _combine_body,
    grid=(N_PAD // RB,),
    in_specs=[
        pl.BlockSpec((RB, D), lambda i: (i, 0)),
        pl.BlockSpec((NC, RB, D), lambda i: (0, i, 0)),
        pl.BlockSpec((RB, 1), lambda i: (i, 0)),
        pl.BlockSpec((1, D), lambda i: (0, 0)),
        pl.BlockSpec((D, D), lambda i: (0, 0)),
    ],
    out_specs=[
        pl.BlockSpec((RB, D), lambda i: (i, 0)),
        pl.BlockSpec((RB, D), lambda i: (i, 0)),
    ],
    out_shape=[
        jax.ShapeDtypeStruct((N_PAD, D), jnp.float32),
        jax.ShapeDtypeStruct((N_PAD, D), jnp.float32),
    ],
)


def _combine3_body(h0_ref, p_ref, deg_ref, bs_ref, r1_ref, r2_ref,
                   r3_ref, m_ref):
    h = _combine_h(h0_ref[...], p_ref[...], deg_ref[...], bs_ref[...])
    r3 = jnp.maximum(h, 0.0)
    r3_ref[...] = r3
    m_ref[...] = jnp.maximum(jnp.maximum(r1_ref[...], r2_ref[...]), r3)


_combine3 = pl.pallas_call(
    _combine3_body,
    grid=(N_PAD // RB,),
    in_specs=[
        pl.BlockSpec((RB, D), lambda i: (i, 0)),
        pl.BlockSpec((NC, RB, D), lambda i: (0, i, 0)),
        pl.BlockSpec((RB, 1), lambda i: (i, 0)),
        pl.BlockSpec((1, D), lambda i: (0, 0)),
        pl.BlockSpec((RB, D), lambda i: (i, 0)),
        pl.BlockSpec((RB, D), lambda i: (i, 0)),
    ],
    out_specs=[
        pl.BlockSpec((RB, D), lambda i: (i, 0)),
        pl.BlockSpec((RB, D), lambda i: (i, 0)),
    ],
    out_shape=[
        jax.ShapeDtypeStruct((N_PAD, D), jnp.float32),
        jax.ShapeDtypeStruct((N_PAD, D), jnp.float32),
    ],
)


def kernel(node_tokens, edge_index, emb, W_c, b_c, W_sum, b_sum):
    tok = jnp.pad(node_tokens.astype(jnp.int32), (0, N_PAD - N))
    tok = tok.reshape(NW, TOK_CH, 80)
    pad_e = E_PAD - E
    src = jnp.concatenate(
        [edge_index[0].astype(jnp.int32), jnp.zeros((pad_e,), jnp.int32)])
    dst = jnp.concatenate(
        [edge_index[1].astype(jnp.int32), jnp.full((pad_e,), N, jnp.int32)])
    src_r = src.reshape(NC, NS, CH, EC)
    dst_r = dst.reshape(NC, NS, CH, EC)
    bc2 = b_c.reshape(1, D)
    bs2 = b_sum.reshape(1, D)

    x = _emb_gather(emb, tok)
    degp = _deg_pass(dst_r)
    deg2 = (degp[0] + degp[1]).reshape(N_PAD, 1)
    h0, t = _mm0(x, W_c, bc2, W_sum)
    p = _edge_pass(t, src_r, dst_r)
    r1, t = _combine(h0, p, deg2, bs2, W_sum)
    p = _edge_pass(t, src_r, dst_r)
    r2, t = _combine(h0, p, deg2, bs2, W_sum)
    p = _edge_pass(t, src_r, dst_r)
    r3, m = _combine3(h0, p, deg2, bs2, r1, r2)

    nl = jnp.stack([r1[:N], r2[:N], r3[:N]])
    return nl, m[:N]
